# baseline (speedup 1.0000x reference)
import numpy as np
import jax
import jax.numpy as jnp
from jax.experimental import pallas as pl
from jax.experimental.pallas import tpu as pltpu

CLAMP = 10.0
BBLK = 2048
CB = 16


def _made_masks(D, H):
    m_in = np.arange(D)
    m0 = np.arange(H) % (D - 1)
    mask1 = (m_in[None, :] <= m0[:, None]).astype(np.float32)
    base = (m0[None, :] < m_in[:, None]).astype(np.float32)
    mask2 = np.repeat(base, 2, axis=0).astype(np.float32)
    return mask1, mask2, m0


def _plan(D, H, sorted_m0):
    blocks = []
    for s in range(0, D, CB):
        n_final = int(np.searchsorted(sorted_m0, s))
        d_al_max = (D + s + CB - 1) // 2
        n_supp = int(np.searchsorted(sorted_m0, d_al_max))
        wlo = (n_final // 128) * 128
        whi = min(H, -(-n_supp // 128) * 128)
        blocks.append((s, wlo, whi))
    return blocks


def _make_body(D, H, blocks):
    def body(z_ref, w1t_ref, b1_ref, wmu_ref, wal_ref, wf_ref, b2f_ref,
             x_ref, ld_ref, acc_a, acc_b, xs_a, xs_b, t_a, t_b):
        Bb = z_ref.shape[0]
        half = Bb // 2
        acc_a[...] = jnp.broadcast_to(b1_ref[...], (half, H))
        acc_b[...] = jnp.broadcast_to(b1_ref[...], (half, H))
        xs_a[...] = jnp.zeros((half, D), jnp.float32)
        xs_b[...] = jnp.zeros((half, D), jnp.float32)
        za = z_ref[0:half, :]
        zb = z_ref[half:Bb, :]
        iota_d = jax.lax.broadcasted_iota(jnp.int32, (1, D), 1)
        iota_f = jax.lax.broadcasted_iota(jnp.int32, (1, 2 * CB), 1)
        lds = (jnp.zeros((half, 1), jnp.float32),
               jnp.zeros((half, 1), jnp.float32))

        for b, (s, wlo, whi) in enumerate(blocks):
            c0 = 2 * CB * b
            Fs = []
            for acc, ts, xs in ((acc_a, t_a, xs_a), (acc_b, t_b, xs_b)):
                if b > 0:
                    prev_s, prev_wlo, prev_whi = blocks[b - 1]
                    if whi > prev_whi:
                        acc[:, prev_whi:whi] = acc[:, prev_whi:whi] + (
                            jnp.dot(xs[:, 0:s], w1t_ref[0:s, prev_whi:whi],
                                    preferred_element_type=jnp.float32))
                    if wlo > prev_wlo:
                        ts[:, prev_wlo:wlo] = jnp.tanh(acc[:, prev_wlo:wlo])
                    F = jnp.dot(ts[:, 0:wlo], wf_ref[0:wlo, c0:c0 + 2 * CB],
                                preferred_element_type=jnp.float32)
                    Fs.append(F + b2f_ref[0:1, c0:c0 + 2 * CB])
                else:
                    Fs.append(b2f_ref[0:1, c0:c0 + 2 * CB])

            def substep(acc, xs, zh, F, oh_mu, oh_al, oh_d, walr, wmur, w1r,
                        b=b, wlo=wlo, whi=whi):
                tw = jnp.tanh(acc[:, wlo:whi])
                al_dyn = jnp.sum(tw * walr, axis=1, keepdims=True)
                mu_f = jnp.sum(F * oh_mu, axis=1, keepdims=True)
                al_f = jnp.sum(F * oh_al, axis=1, keepdims=True)
                if b == 0:
                    mu_dyn = jnp.sum(tw[:, 0:128] * wmur, axis=1,
                                     keepdims=True)
                    mu = jnp.clip(mu_f + mu_dyn, -CLAMP, CLAMP)
                else:
                    mu = jnp.clip(mu_f, -CLAMP, CLAMP)
                al = jnp.clip(al_f + al_dyn, -CLAMP, CLAMP)
                z_i = jnp.sum(zh * oh_d, axis=1, keepdims=True)
                x_i = z_i * jnp.exp(al) + mu
                acc[:, wlo:whi] = acc[:, wlo:whi] + x_i * w1r
                xs[...] = xs[...] + x_i * oh_d
                return al

            def step(i, lds, Fa=Fs[0], Fb=Fs[1], s=s, wlo=wlo, whi=whi):
                ld_a, ld_b = lds
                oh_mu = (iota_f == (i - s)).astype(jnp.float32)
                oh_al = (iota_f == (CB + i - s)).astype(jnp.float32)
                oh_d = (iota_d == i).astype(jnp.float32)
                walr = wal_ref[pl.ds(i, 1), :][:, wlo:whi]
                wmur = wmu_ref[pl.ds(i, 1), :][:, 0:128]
                w1r = w1t_ref[pl.ds(i, 1), :][:, wlo:whi]
                al_a = substep(acc_a, xs_a, za, Fa, oh_mu, oh_al, oh_d,
                               walr, wmur, w1r)
                al_b = substep(acc_b, xs_b, zb, Fb, oh_mu, oh_al, oh_d,
                               walr, wmur, w1r)
                return (ld_a + al_a, ld_b + al_b)

            def oct8(k, c, s=s):
                for j in range(8):
                    c = step(s + 8 * k + j, c)
                return c

            lds = jax.lax.fori_loop(0, CB // 8, oct8, lds)

        xa = xs_a[...]
        xb = xs_b[...]
        x_ref[0:half, :] = jnp.where(jnp.isnan(xa) | jnp.isinf(xa), 0.0, xa)
        x_ref[half:Bb, :] = jnp.where(jnp.isnan(xb) | jnp.isinf(xb), 0.0, xb)
        ld_a, ld_b = lds
        ld_ref[0:half, :] = jnp.where(jnp.isnan(ld_a) | jnp.isinf(ld_a),
                                      0.0, ld_a)
        ld_ref[half:Bb, :] = jnp.where(jnp.isnan(ld_b) | jnp.isinf(ld_b),
                                       0.0, ld_b)
    return body


def kernel(z, W1, b1, W2, b2):
    B, D = z.shape
    H = W1.shape[0]
    mask1, mask2, m0 = _made_masks(D, H)
    perm = np.argsort(m0, kind="stable")
    sorted_m0 = m0[perm]
    blocks = _plan(D, H, sorted_m0)

    perm_mat = np.zeros((H, H), np.float32)
    perm_mat[perm, np.arange(H)] = 1.0
    P = jnp.asarray(perm_mat)
    w1t = (W1 * mask1).T @ P
    W2m = W2 * mask2
    wmu = W2m[:D] @ P
    wal = W2m[D:] @ P
    b1r = b1.reshape(1, H) @ P
    b2mu = b2[:D]
    b2al = b2[D:]

    wf_cols, b2f_cols = [], []
    for s in range(0, D, CB):
        wf_cols += [wmu[s:s + CB].T, wal[s:s + CB].T]
        b2f_cols += [b2mu[s:s + CB], b2al[s:s + CB]]
    wf = jnp.concatenate(wf_cols, axis=1)
    b2f = jnp.concatenate(b2f_cols).reshape(1, -1)

    bblk = min(BBLK, B)
    half = bblk // 2
    x, ld = pl.pallas_call(
        _make_body(D, H, blocks),
        grid=(B // bblk,),
        in_specs=[
            pl.BlockSpec((bblk, D), lambda i: (i, 0)),
            pl.BlockSpec((D, H), lambda i: (0, 0)),
            pl.BlockSpec((1, H), lambda i: (0, 0)),
            pl.BlockSpec((D, H), lambda i: (0, 0)),
            pl.BlockSpec((D, H), lambda i: (0, 0)),
            pl.BlockSpec((H, wf.shape[1]), lambda i: (0, 0)),
            pl.BlockSpec((1, b2f.shape[1]), lambda i: (0, 0)),
        ],
        out_specs=[
            pl.BlockSpec((bblk, D), lambda i: (i, 0)),
            pl.BlockSpec((bblk, 1), lambda i: (i, 0)),
        ],
        out_shape=[
            jax.ShapeDtypeStruct((B, D), jnp.float32),
            jax.ShapeDtypeStruct((B, 1), jnp.float32),
        ],
        scratch_shapes=[
            pltpu.VMEM((half, H), jnp.float32),
            pltpu.VMEM((half, H), jnp.float32),
            pltpu.VMEM((half, D), jnp.float32),
            pltpu.VMEM((half, D), jnp.float32),
            pltpu.VMEM((half, H), jnp.float32),
            pltpu.VMEM((half, H), jnp.float32),
        ],
        compiler_params=pltpu.CompilerParams(
            dimension_semantics=("parallel",),
        ),
    )(z, w1t, b1r, wmu, wal, wf, b2f)
    return x, ld.reshape(B)


# speedup vs baseline: 1.1299x; 1.1299x over previous
import numpy as np
import jax
import jax.numpy as jnp
from jax.experimental import pallas as pl
from jax.experimental.pallas import tpu as pltpu

CLAMP = 10.0
BBLK = 2048
CB = 16


def _made_masks(D, H):
    m_in = np.arange(D)
    m0 = np.arange(H) % (D - 1)
    mask1 = (m_in[None, :] <= m0[:, None]).astype(np.float32)
    base = (m0[None, :] < m_in[:, None]).astype(np.float32)
    mask2 = np.repeat(base, 2, axis=0).astype(np.float32)
    return mask1, mask2, m0


def _plan(D, H, sorted_m0):
    blocks = []
    for s in range(0, D, CB):
        n_final = int(np.searchsorted(sorted_m0, s))
        d_al_max = (D + s + CB - 1) // 2
        n_supp = int(np.searchsorted(sorted_m0, d_al_max))
        wlo = (n_final // 128) * 128
        whi = min(H, -(-n_supp // 128) * 128)
        blocks.append((s, wlo, whi))
    return blocks


def _make_body(D, H, blocks):
    def body(z_ref, w1t_ref, b1_ref, wmu_ref, wal_ref, wf_ref, b2f_ref,
             x_ref, ld_ref, acc_a, acc_b, xs_a, xs_b, t_a, t_b):
        Bb = z_ref.shape[0]
        half = Bb // 2
        acc_a[...] = jnp.broadcast_to(b1_ref[...], (half, H))
        acc_b[...] = jnp.broadcast_to(b1_ref[...], (half, H))
        xs_a[...] = jnp.zeros((half, D), jnp.float32)
        xs_b[...] = jnp.zeros((half, D), jnp.float32)
        za = z_ref[0:half, :]
        zb = z_ref[half:Bb, :]
        iota_d = jax.lax.broadcasted_iota(jnp.int32, (1, D), 1)
        iota_f = jax.lax.broadcasted_iota(jnp.int32, (1, 2 * CB), 1)
        lds = (jnp.zeros((half, 1), jnp.float32),
               jnp.zeros((half, 1), jnp.float32))

        for b, (s, wlo, whi) in enumerate(blocks):
            c0 = 2 * CB * b
            Fs = []
            for acc, ts, xs in ((acc_a, t_a, xs_a), (acc_b, t_b, xs_b)):
                if b > 0:
                    prev_s, prev_wlo, prev_whi = blocks[b - 1]
                    if whi > prev_whi:
                        acc[:, prev_whi:whi] = acc[:, prev_whi:whi] + (
                            jnp.dot(xs[:, 0:s], w1t_ref[0:s, prev_whi:whi],
                                    preferred_element_type=jnp.float32))
                    if wlo > prev_wlo:
                        ts[:, prev_wlo:wlo] = jnp.tanh(acc[:, prev_wlo:wlo])
                    F = jnp.dot(ts[:, 0:wlo], wf_ref[0:wlo, c0:c0 + 2 * CB],
                                preferred_element_type=jnp.float32)
                    Fs.append(F + b2f_ref[0:1, c0:c0 + 2 * CB])
                else:
                    Fs.append(b2f_ref[0:1, c0:c0 + 2 * CB])

            def substep(acc, xs, zh, F, oh_mu, oh_al, oh_d, walr, wmur, w1r,
                        b=b, wlo=wlo, whi=whi):
                tw = jnp.tanh(acc[:, wlo:whi])
                al_dyn = jnp.sum(tw * walr, axis=1, keepdims=True)
                mu_f = jnp.sum(F * oh_mu, axis=1, keepdims=True)
                al_f = jnp.sum(F * oh_al, axis=1, keepdims=True)
                if b == 0:
                    mu_dyn = jnp.sum(tw[:, 0:128] * wmur, axis=1,
                                     keepdims=True)
                    mu = jnp.clip(mu_f + mu_dyn, -CLAMP, CLAMP)
                else:
                    mu = jnp.clip(mu_f, -CLAMP, CLAMP)
                al = jnp.clip(al_f + al_dyn, -CLAMP, CLAMP)
                z_i = jnp.sum(zh * oh_d, axis=1, keepdims=True)
                x_i = z_i * jnp.exp(al) + mu
                acc[:, wlo:whi] = acc[:, wlo:whi] + x_i * w1r
                xs[...] = xs[...] + x_i * oh_d
                return al

            def step(i, lds, Fa=Fs[0], Fb=Fs[1], s=s, wlo=wlo, whi=whi):
                ld_a, ld_b = lds
                oh_mu = (iota_f == (i - s)).astype(jnp.float32)
                oh_al = (iota_f == (CB + i - s)).astype(jnp.float32)
                oh_d = (iota_d == i).astype(jnp.float32)
                walr = wal_ref[pl.ds(i, 1), :][:, wlo:whi]
                wmur = wmu_ref[pl.ds(i, 1), :][:, 0:128]
                w1r = w1t_ref[pl.ds(i, 1), :][:, wlo:whi]
                al_a = substep(acc_a, xs_a, za, Fa, oh_mu, oh_al, oh_d,
                               walr, wmur, w1r)
                al_b = substep(acc_b, xs_b, zb, Fb, oh_mu, oh_al, oh_d,
                               walr, wmur, w1r)
                return (ld_a + al_a, ld_b + al_b)

            def quad(k, c, s=s):
                c = step(s + 4 * k + 1, step(s + 4 * k, c))
                return step(s + 4 * k + 3, step(s + 4 * k + 2, c))

            lds = jax.lax.fori_loop(0, CB // 4, quad, lds)

        xa = xs_a[...]
        xb = xs_b[...]
        x_ref[0:half, :] = jnp.where(jnp.isnan(xa) | jnp.isinf(xa), 0.0, xa)
        x_ref[half:Bb, :] = jnp.where(jnp.isnan(xb) | jnp.isinf(xb), 0.0, xb)
        ld_a, ld_b = lds
        ld_ref[0:half, :] = jnp.where(jnp.isnan(ld_a) | jnp.isinf(ld_a),
                                      0.0, ld_a)
        ld_ref[half:Bb, :] = jnp.where(jnp.isnan(ld_b) | jnp.isinf(ld_b),
                                       0.0, ld_b)
    return body


def kernel(z, W1, b1, W2, b2):
    B, D = z.shape
    H = W1.shape[0]
    mask1, mask2, m0 = _made_masks(D, H)
    perm = np.argsort(m0, kind="stable")
    sorted_m0 = m0[perm]
    blocks = _plan(D, H, sorted_m0)

    perm_mat = np.zeros((H, H), np.float32)
    perm_mat[perm, np.arange(H)] = 1.0
    P = jnp.asarray(perm_mat)
    w1t = (W1 * mask1).T @ P
    W2m = W2 * mask2
    wmu = W2m[:D] @ P
    wal = W2m[D:] @ P
    b1r = b1.reshape(1, H) @ P
    b2mu = b2[:D]
    b2al = b2[D:]

    wf_cols, b2f_cols = [], []
    for s in range(0, D, CB):
        wf_cols += [wmu[s:s + CB].T, wal[s:s + CB].T]
        b2f_cols += [b2mu[s:s + CB], b2al[s:s + CB]]
    wf = jnp.concatenate(wf_cols, axis=1)
    b2f = jnp.concatenate(b2f_cols).reshape(1, -1)

    bblk = min(BBLK, B)
    half = bblk // 2
    x, ld = pl.pallas_call(
        _make_body(D, H, blocks),
        grid=(B // bblk,),
        in_specs=[
            pl.BlockSpec((bblk, D), lambda i: (i, 0)),
            pl.BlockSpec((D, H), lambda i: (0, 0)),
            pl.BlockSpec((1, H), lambda i: (0, 0)),
            pl.BlockSpec((D, H), lambda i: (0, 0)),
            pl.BlockSpec((D, H), lambda i: (0, 0)),
            pl.BlockSpec((H, wf.shape[1]), lambda i: (0, 0)),
            pl.BlockSpec((1, b2f.shape[1]), lambda i: (0, 0)),
        ],
        out_specs=[
            pl.BlockSpec((bblk, D), lambda i: (i, 0)),
            pl.BlockSpec((bblk, 1), lambda i: (i, 0)),
        ],
        out_shape=[
            jax.ShapeDtypeStruct((B, D), jnp.float32),
            jax.ShapeDtypeStruct((B, 1), jnp.float32),
        ],
        scratch_shapes=[
            pltpu.VMEM((half, H), jnp.float32),
            pltpu.VMEM((half, H), jnp.float32),
            pltpu.VMEM((half, D), jnp.float32),
            pltpu.VMEM((half, D), jnp.float32),
            pltpu.VMEM((half, H), jnp.float32),
            pltpu.VMEM((half, H), jnp.float32),
        ],
        compiler_params=pltpu.CompilerParams(
            dimension_semantics=("parallel",),
        ),
    )(z, w1t, b1r, wmu, wal, wf, b2f)
    return x, ld.reshape(B)
